# baseline (speedup 1.0000x reference)
import jax
import jax.numpy as jnp
from jax.experimental import pallas as pl
from jax.experimental.pallas import tpu as pltpu
from functools import partial

N_BINS = 32
EPS = 6.4 / 256
LOW = -3.2 - EPS / 2
HIGH = 3.2 - EPS / 2
WIDTH = HIGH - LOW


def _downsample_kernel(x_ref, out_ref):
    x = x_ref[:, 0, 0, :]
    rows = jax.lax.broadcasted_iota(jnp.int32, (512, N_BINS), 0)
    cols = jax.lax.broadcasted_iota(jnp.int32, (512, N_BINS), 1)
    sel = (rows == cols * 16).astype(jnp.float32)
    out_ref[:, 0, 0, :] = jax.lax.dot_general(
        x, sel, (((1,), (0,)), ((), ())), preferred_element_type=jnp.float32,
        precision=jax.lax.Precision.HIGHEST)


def _hist_mlp_kernel(pix_ref, w1_ref, b1_ref, w2_ref, b2_ref,
                     w3_ref, b3_ref, out_ref):
    p0 = pix_ref[:, 0, :]
    p1 = pix_ref[:, 1, :]
    p2 = pix_ref[:, 2, :]
    B, P = p0.shape

    valid = ((p0 > 0) & (p1 > 0) & (p2 > 0)).astype(jnp.float32)
    iy = jnp.sqrt(p0 * p0 + p1 * p1 + p2 * p2)
    s0 = jnp.where(p0 > 0, p0, 1.0)
    s1 = jnp.where(p1 > 0, p1, 1.0)
    s2 = jnp.where(p2 > 0, p2, 1.0)

    base_w = iy * valid
    bins = jax.lax.broadcasted_iota(jnp.int32, (B, P, N_BINS), 2)

    hist_parts = []
    for (si, su, sv) in ((s0, s2, s1), (s1, s2, s0), (s2, s1, s0)):
        iu = jnp.log(si / su)
        iv = jnp.log(si / sv)
        bu = jnp.floor((iu - LOW) / WIDTH * N_BINS).astype(jnp.int32)
        bv = jnp.floor((iv - LOW) / WIDTH * N_BINS).astype(jnp.int32)
        bu = jnp.where(iu == HIGH, N_BINS - 1, bu)
        bv = jnp.where(iv == HIGH, N_BINS - 1, bv)
        in_u = ((iu >= LOW) & (iu <= HIGH) & (bu >= 0) & (bu < N_BINS))
        in_v = ((iv >= LOW) & (iv <= HIGH) & (bv >= 0) & (bv < N_BINS))
        w = base_w * in_u.astype(jnp.float32) * in_v.astype(jnp.float32)

        u_oh = jnp.where(bu[:, :, None] == bins, w[:, :, None], 0.0)
        v_oh = jnp.where(bv[:, :, None] == bins, 1.0, 0.0)
        h2d = jax.lax.dot_general(
            u_oh, v_oh,
            dimension_numbers=(((1,), (1,)), ((0,), (0,))),
            preferred_element_type=jnp.float32,
        )
        h = h2d.reshape(B, N_BINS * N_BINS)
        norm = jnp.sum(h, axis=1, keepdims=True)
        hist_parts.append(jnp.sqrt(h / norm))

    hist = jnp.concatenate(hist_parts, axis=1)

    h1 = jax.lax.dot_general(hist, w1_ref[...], (((1,), (1,)), ((), ())),
                             preferred_element_type=jnp.float32)
    h1 = jnp.maximum(h1 + b1_ref[...][None, :], 0.0)
    h2 = jax.lax.dot_general(h1, w2_ref[...], (((1,), (1,)), ((), ())),
                             preferred_element_type=jnp.float32)
    h2 = jnp.maximum(h2 + b2_ref[...][None, :], 0.0)
    h3 = jax.lax.dot_general(h2, w3_ref[...], (((1,), (1,)), ((), ())),
                             preferred_element_type=jnp.float32)
    out_ref[...] = jnp.maximum(h3 + b3_ref[...][None, :], 0.0)


@jax.jit
def kernel(inp_img, W1, b1, W2, b2, W3, b3):
    B, C, H, W = inp_img.shape
    stride = H // N_BINS
    x4 = inp_img.reshape(B * C, H, 1, W)

    small = pl.pallas_call(
        _downsample_kernel,
        grid=(N_BINS,),
        in_specs=[pl.BlockSpec((B * C, 1, 1, W), lambda i: (0, stride * i, 0, 0))],
        out_specs=pl.BlockSpec((B * C, 1, 1, N_BINS), lambda i: (0, i, 0, 0)),
        out_shape=jax.ShapeDtypeStruct((B * C, N_BINS, 1, N_BINS), jnp.float32),
    )(x4)
    pix = small.reshape(B, C, N_BINS * N_BINS)

    out = pl.pallas_call(
        _hist_mlp_kernel,
        out_shape=jax.ShapeDtypeStruct((B, W3.shape[0]), jnp.float32),
        compiler_params=pltpu.CompilerParams(
            vmem_limit_bytes=100 * 1024 * 1024,
        ),
    )(pix, W1, b1, W2, b2, W3, b3)
    return out[:, :, None, None]


# speedup vs baseline: 4.3422x; 4.3422x over previous
import jax
import jax.numpy as jnp
from jax.experimental import pallas as pl
from jax.experimental.pallas import tpu as pltpu
from functools import partial

N_BINS = 32
EPS = 6.4 / 256
LOW = -3.2 - EPS / 2
HIGH = 3.2 - EPS / 2
WIDTH = HIGH - LOW


def _downsample_kernel(x_ref, out_ref):
    x = x_ref[:, 0, :]
    rows = jax.lax.broadcasted_iota(jnp.int32, (512, N_BINS), 0)
    cols = jax.lax.broadcasted_iota(jnp.int32, (512, N_BINS), 1)
    sel = (rows == cols * 16).astype(jnp.float32)
    out_ref[0, :, :] = jax.lax.dot_general(
        x, sel, (((1,), (0,)), ((), ())), preferred_element_type=jnp.float32,
        precision=jax.lax.Precision.HIGHEST)


def _hist_mlp_kernel(pix_ref, w1_ref, b1_ref, w2_ref, b2_ref,
                     w3_ref, b3_ref, out_ref):
    p0 = pix_ref[:, 0, :]
    p1 = pix_ref[:, 1, :]
    p2 = pix_ref[:, 2, :]
    B, P = p0.shape

    valid = ((p0 > 0) & (p1 > 0) & (p2 > 0)).astype(jnp.float32)
    iy = jnp.sqrt(p0 * p0 + p1 * p1 + p2 * p2)
    s0 = jnp.where(p0 > 0, p0, 1.0)
    s1 = jnp.where(p1 > 0, p1, 1.0)
    s2 = jnp.where(p2 > 0, p2, 1.0)

    base_w = iy * valid
    bins = jax.lax.broadcasted_iota(jnp.int32, (B, P, N_BINS), 2)

    hist_parts = []
    for (si, su, sv) in ((s0, s2, s1), (s1, s2, s0), (s2, s1, s0)):
        iu = jnp.log(si / su)
        iv = jnp.log(si / sv)
        bu = jnp.floor((iu - LOW) / WIDTH * N_BINS).astype(jnp.int32)
        bv = jnp.floor((iv - LOW) / WIDTH * N_BINS).astype(jnp.int32)
        bu = jnp.where(iu == HIGH, N_BINS - 1, bu)
        bv = jnp.where(iv == HIGH, N_BINS - 1, bv)
        in_u = ((iu >= LOW) & (iu <= HIGH) & (bu >= 0) & (bu < N_BINS))
        in_v = ((iv >= LOW) & (iv <= HIGH) & (bv >= 0) & (bv < N_BINS))
        w = base_w * in_u.astype(jnp.float32) * in_v.astype(jnp.float32)

        u_oh = jnp.where(bu[:, :, None] == bins, w[:, :, None], 0.0)
        v_oh = jnp.where(bv[:, :, None] == bins, 1.0, 0.0)
        h2d = jax.lax.dot_general(
            u_oh, v_oh,
            dimension_numbers=(((1,), (1,)), ((0,), (0,))),
            preferred_element_type=jnp.float32,
        )
        h = h2d.reshape(B, N_BINS * N_BINS)
        norm = jnp.sum(h, axis=1, keepdims=True)
        hist_parts.append(jnp.sqrt(h / norm))

    hist = jnp.concatenate(hist_parts, axis=1)

    h1 = jax.lax.dot_general(hist, w1_ref[...], (((1,), (1,)), ((), ())),
                             preferred_element_type=jnp.float32)
    h1 = jnp.maximum(h1 + b1_ref[...][None, :], 0.0)
    h2 = jax.lax.dot_general(h1, w2_ref[...], (((1,), (1,)), ((), ())),
                             preferred_element_type=jnp.float32)
    h2 = jnp.maximum(h2 + b2_ref[...][None, :], 0.0)
    h3 = jax.lax.dot_general(h2, w3_ref[...], (((1,), (1,)), ((), ())),
                             preferred_element_type=jnp.float32)
    out_ref[...] = jnp.maximum(h3 + b3_ref[...][None, :], 0.0)


@jax.jit
def kernel(inp_img, W1, b1, W2, b2, W3, b3):
    B, C, H, W = inp_img.shape
    stride = H // N_BINS
    x3 = inp_img.reshape(B * C, H, W)

    small_t = pl.pallas_call(
        _downsample_kernel,
        grid=(N_BINS,),
        in_specs=[pl.BlockSpec((B * C, 8, W), lambda i: (0, 2 * i, 0))],
        out_specs=pl.BlockSpec((1, B * C, N_BINS), lambda i: (i, 0, 0)),
        out_shape=jax.ShapeDtypeStruct((N_BINS, B * C, N_BINS), jnp.float32),
    )(x3)
    pix = small_t.transpose(1, 0, 2).reshape(B, C, N_BINS * N_BINS)

    out = pl.pallas_call(
        _hist_mlp_kernel,
        out_shape=jax.ShapeDtypeStruct((B, W3.shape[0]), jnp.float32),
        compiler_params=pltpu.CompilerParams(
            vmem_limit_bytes=100 * 1024 * 1024,
        ),
    )(pix, W1, b1, W2, b2, W3, b3)
    return out[:, :, None, None]


# speedup vs baseline: 5.2031x; 1.1983x over previous
import jax
import jax.numpy as jnp
from jax import lax
from jax.experimental import pallas as pl
from jax.experimental.pallas import tpu as pltpu
from jax.experimental.pallas import tpu_sc as plsc
from functools import partial

N_BINS = 32
EPS = 6.4 / 256
LOW = -3.2 - EPS / 2
HIGH = 3.2 - EPS / 2
WIDTH = HIGH - LOW

NUM_WORKERS = 32
ROWS_PER_WORKER = 3


def _bin_one(si, su, sv, base_w):
    iu = jnp.log(si / su)
    iv = jnp.log(si / sv)
    bu = jnp.floor((iu - LOW) / WIDTH * N_BINS).astype(jnp.int32)
    bv = jnp.floor((iv - LOW) / WIDTH * N_BINS).astype(jnp.int32)
    bu = jnp.where(iu == HIGH, N_BINS - 1, bu)
    bv = jnp.where(iv == HIGH, N_BINS - 1, bv)
    in_u = ((iu >= LOW) & (iu <= HIGH) & (bu >= 0) & (bu < N_BINS))
    in_v = ((iv >= LOW) & (iv <= HIGH) & (bv >= 0) & (bv < N_BINS))
    w = base_w * in_u.astype(jnp.float32) * in_v.astype(jnp.float32)
    idx = jnp.where(w > 0, bu * N_BINS + bv, 0)
    return idx, w


def _gather_bin_kernel(x_ref, idx_ref, w_ref):
    x = x_ref[:, 0, :]
    rows = lax.broadcasted_iota(jnp.int32, (512, N_BINS), 0)
    cols = lax.broadcasted_iota(jnp.int32, (512, N_BINS), 1)
    sel = (rows == cols * 16).astype(jnp.float32)
    y = lax.dot_general(x, sel, (((1,), (0,)), ((), ())),
                        preferred_element_type=jnp.float32,
                        precision=lax.Precision.HIGHEST)
    p = y.reshape(32, 3, N_BINS)
    p0, p1, p2 = p[:, 0, :], p[:, 1, :], p[:, 2, :]

    valid = ((p0 > 0) & (p1 > 0) & (p2 > 0)).astype(jnp.float32)
    iy = jnp.sqrt(p0 * p0 + p1 * p1 + p2 * p2)
    s0 = jnp.where(p0 > 0, p0, 1.0)
    s1 = jnp.where(p1 > 0, p1, 1.0)
    s2 = jnp.where(p2 > 0, p2, 1.0)
    base_w = iy * valid

    i0, w0 = _bin_one(s0, s2, s1, base_w)
    i1, w1 = _bin_one(s1, s2, s0, base_w)
    i2, w2 = _bin_one(s2, s1, s0, base_w)

    idx_ref[0, :, :] = jnp.stack([i0, i1, i2], axis=1).reshape(96, N_BINS)
    w_ref[0, :, :] = jnp.stack([w0, w1, w2], axis=1).reshape(96, N_BINS)


def _sc_scatter_body(idx_hbm, w_hbm, out_hbm, idx_v, w_v, hist_v):
    c = lax.axis_index("c")
    s = lax.axis_index("s")
    wid = s * 2 + c
    n = ROWS_PER_WORKER * N_BINS * N_BINS
    base = wid * n
    pltpu.sync_copy(idx_hbm.at[pl.ds(base, n)], idx_v)
    pltpu.sync_copy(w_hbm.at[pl.ds(base, n)], w_v)

    zeros = jnp.zeros((16,), jnp.float32)

    def zero_body(k, carry):
        hist_v[pl.ds(k * 16, 16)] = zeros
        return carry

    lax.fori_loop(0, n // 16, zero_body, 0)

    def scatter_body(k, carry):
        iv = idx_v[pl.ds(k * 16, 16)]
        wv = w_v[pl.ds(k * 16, 16)]
        row_base = (k // 64) * (N_BINS * N_BINS)
        plsc.addupdate_scatter(hist_v, [iv + row_base], wv)
        return carry

    lax.fori_loop(0, n // 16, scatter_body, 0)

    pltpu.sync_copy(hist_v, out_hbm.at[pl.ds(base, n)])


def _norm_mlp_kernel(h_ref, w1_ref, b1_ref, w2_ref, b2_ref, w3_ref, b3_ref,
                     out_ref):
    h = h_ref[...]
    P2 = N_BINS * N_BINS
    parts = []
    for c in range(3):
        hc = h[:, c * P2:(c + 1) * P2]
        norm = jnp.sum(hc, axis=1, keepdims=True)
        parts.append(jnp.sqrt(hc / norm))
    hist = jnp.concatenate(parts, axis=1)

    h1 = lax.dot_general(hist, w1_ref[...], (((1,), (1,)), ((), ())),
                         preferred_element_type=jnp.float32)
    h1 = jnp.maximum(h1 + b1_ref[...][None, :], 0.0)
    h2 = lax.dot_general(h1, w2_ref[...], (((1,), (1,)), ((), ())),
                         preferred_element_type=jnp.float32)
    h2 = jnp.maximum(h2 + b2_ref[...][None, :], 0.0)
    h3 = lax.dot_general(h2, w3_ref[...], (((1,), (1,)), ((), ())),
                         preferred_element_type=jnp.float32)
    out_ref[...] = jnp.maximum(h3 + b3_ref[...][None, :], 0.0)


@jax.jit
def kernel(inp_img, W1, b1, W2, b2, W3, b3):
    B, C, H, W = inp_img.shape
    BC = B * C
    x3 = inp_img.reshape(BC, H, W)

    idx_t, w_t = pl.pallas_call(
        _gather_bin_kernel,
        grid=(N_BINS,),
        in_specs=[pl.BlockSpec((BC, 8, W), lambda i: (0, 2 * i, 0))],
        out_specs=[
            pl.BlockSpec((1, BC, N_BINS), lambda i: (i, 0, 0)),
            pl.BlockSpec((1, BC, N_BINS), lambda i: (i, 0, 0)),
        ],
        out_shape=[
            jax.ShapeDtypeStruct((N_BINS, BC, N_BINS), jnp.int32),
            jax.ShapeDtypeStruct((N_BINS, BC, N_BINS), jnp.float32),
        ],
    )(x3)

    P2 = N_BINS * N_BINS
    idx_flat = idx_t.transpose(1, 0, 2).reshape(BC * P2)
    w_flat = w_t.transpose(1, 0, 2).reshape(BC * P2)

    npw = ROWS_PER_WORKER * P2
    sc_scatter = pl.kernel(
        _sc_scatter_body,
        out_type=jax.ShapeDtypeStruct((BC * P2,), jnp.float32),
        scratch_types=[
            pltpu.VMEM((npw,), jnp.int32),
            pltpu.VMEM((npw,), jnp.float32),
            pltpu.VMEM((npw,), jnp.float32),
        ],
        mesh=plsc.VectorSubcoreMesh(core_axis_name="c", subcore_axis_name="s"),
        compiler_params=pltpu.CompilerParams(needs_layout_passes=False),
    )
    hist96 = sc_scatter(idx_flat, w_flat)

    hist32 = hist96.reshape(B, C * P2)

    out = pl.pallas_call(
        _norm_mlp_kernel,
        out_shape=jax.ShapeDtypeStruct((B, W3.shape[0]), jnp.float32),
        compiler_params=pltpu.CompilerParams(
            vmem_limit_bytes=100 * 1024 * 1024,
        ),
    )(hist32, W1, b1, W2, b2, W3, b3)
    return out[:, :, None, None]


# speedup vs baseline: 5.2892x; 1.0165x over previous
import jax
import jax.numpy as jnp
from jax import lax
from jax.experimental import pallas as pl
from jax.experimental.pallas import tpu as pltpu
from jax.experimental.pallas import tpu_sc as plsc
from functools import partial

N_BINS = 32
EPS = 6.4 / 256
LOW = -3.2 - EPS / 2
HIGH = 3.2 - EPS / 2
WIDTH = HIGH - LOW

NUM_WORKERS = 32
ROWS_PER_WORKER = 3


def _bin_one(si, su, sv, base_w):
    iu = jnp.log(si / su)
    iv = jnp.log(si / sv)
    bu = jnp.floor((iu - LOW) / WIDTH * N_BINS).astype(jnp.int32)
    bv = jnp.floor((iv - LOW) / WIDTH * N_BINS).astype(jnp.int32)
    bu = jnp.where(iu == HIGH, N_BINS - 1, bu)
    bv = jnp.where(iv == HIGH, N_BINS - 1, bv)
    in_u = ((iu >= LOW) & (iu <= HIGH) & (bu >= 0) & (bu < N_BINS))
    in_v = ((iv >= LOW) & (iv <= HIGH) & (bv >= 0) & (bv < N_BINS))
    w = base_w * in_u.astype(jnp.float32) * in_v.astype(jnp.float32)
    idx = jnp.where(w > 0, bu * N_BINS + bv, 0)
    return idx, w


def _gather_bin_kernel(x_ref, idx_ref, w_ref):
    x = x_ref[:, 0, :]
    rows = lax.broadcasted_iota(jnp.int32, (512, N_BINS), 0)
    cols = lax.broadcasted_iota(jnp.int32, (512, N_BINS), 1)
    sel = (rows == cols * 16).astype(jnp.float32)
    y = lax.dot_general(x, sel, (((1,), (0,)), ((), ())),
                        preferred_element_type=jnp.float32,
                        precision=lax.Precision.HIGHEST)
    p = y.reshape(32, 3, N_BINS)
    p0, p1, p2 = p[:, 0, :], p[:, 1, :], p[:, 2, :]

    valid = ((p0 > 0) & (p1 > 0) & (p2 > 0)).astype(jnp.float32)
    iy = jnp.sqrt(p0 * p0 + p1 * p1 + p2 * p2)
    s0 = jnp.where(p0 > 0, p0, 1.0)
    s1 = jnp.where(p1 > 0, p1, 1.0)
    s2 = jnp.where(p2 > 0, p2, 1.0)
    base_w = iy * valid

    i0, w0 = _bin_one(s0, s2, s1, base_w)
    i1, w1 = _bin_one(s1, s2, s0, base_w)
    i2, w2 = _bin_one(s2, s1, s0, base_w)

    j = pl.program_id(1)
    vi = jnp.stack([i0, i1, i2], axis=1).reshape(96, N_BINS)
    vw = jnp.stack([w0, w1, w2], axis=1).reshape(96, N_BINS)
    for jj in range(4):
        @pl.when(j == jj)
        def _(jj=jj):
            idx_ref[:, jj * N_BINS:(jj + 1) * N_BINS] = vi
            w_ref[:, jj * N_BINS:(jj + 1) * N_BINS] = vw


def _sc_scatter_body(idx_hbm, w_hbm, out_hbm, idx_v, w_v, hist_v):
    c = lax.axis_index("c")
    s = lax.axis_index("s")
    P2 = N_BINS * N_BINS

    @pl.when(s < 6)
    def _():
        base = (c * 6 + s) * 8
        pltpu.sync_copy(idx_hbm.at[pl.ds(base, 8), :], idx_v)
        pltpu.sync_copy(w_hbm.at[pl.ds(base, 8), :], w_v)

        zeros = jnp.zeros((16,), jnp.float32)

        def zero_body(k, carry):
            for j in range(8):
                hist_v[j, pl.ds(k * 16, 16)] = zeros
            return carry

        lax.fori_loop(0, P2 // 16, zero_body, 0)

        def scatter_body(k, carry):
            j = k // (P2 // 16)
            rowv = jnp.full((16,), j, jnp.int32)
            iv = idx_v[j, pl.ds((k % 64) * 16, 16)]
            wv = w_v[j, pl.ds((k % 64) * 16, 16)]
            plsc.addupdate_scatter(hist_v, [rowv, iv], wv)
            return carry

        lax.fori_loop(0, 8 * P2 // 16, scatter_body, 0)

        pltpu.sync_copy(hist_v, out_hbm.at[pl.ds(base, 8), :])


def _norm_mlp_kernel(h_ref, w1_ref, b1_ref, w2_ref, b2_ref, w3_ref, b3_ref,
                     out_ref):
    h = h_ref[...]
    P2 = N_BINS * N_BINS
    parts = []
    for c in range(3):
        hc = h[:, c * P2:(c + 1) * P2]
        norm = jnp.sum(hc, axis=1, keepdims=True)
        parts.append(jnp.sqrt(hc / norm))
    hist = jnp.concatenate(parts, axis=1)

    h1 = lax.dot_general(hist, w1_ref[...], (((1,), (1,)), ((), ())),
                         preferred_element_type=jnp.float32)
    h1 = jnp.maximum(h1 + b1_ref[...][None, :], 0.0)
    h2 = lax.dot_general(h1, w2_ref[...], (((1,), (1,)), ((), ())),
                         preferred_element_type=jnp.float32)
    h2 = jnp.maximum(h2 + b2_ref[...][None, :], 0.0)
    h3 = lax.dot_general(h2, w3_ref[...], (((1,), (1,)), ((), ())),
                         preferred_element_type=jnp.float32)
    out_ref[...] = jnp.maximum(h3 + b3_ref[...][None, :], 0.0)


@jax.jit
def kernel(inp_img, W1, b1, W2, b2, W3, b3):
    B, C, H, W = inp_img.shape
    BC = B * C
    x3 = inp_img.reshape(BC, H, W)

    P2 = N_BINS * N_BINS
    idx96, w96 = pl.pallas_call(
        _gather_bin_kernel,
        grid=(8, 4),
        in_specs=[pl.BlockSpec((BC, 8, W), lambda i, j: (0, 2 * (4 * i + j), 0))],
        out_specs=[
            pl.BlockSpec((BC, 4 * N_BINS), lambda i, j: (0, i)),
            pl.BlockSpec((BC, 4 * N_BINS), lambda i, j: (0, i)),
        ],
        out_shape=[
            jax.ShapeDtypeStruct((BC, P2), jnp.int32),
            jax.ShapeDtypeStruct((BC, P2), jnp.float32),
        ],
    )(x3)

    sc_scatter = pl.kernel(
        _sc_scatter_body,
        out_type=jax.ShapeDtypeStruct((BC, P2), jnp.float32),
        scratch_types=[
            pltpu.VMEM((8, P2), jnp.int32),
            pltpu.VMEM((8, P2), jnp.float32),
            pltpu.VMEM((8, P2), jnp.float32),
        ],
        mesh=plsc.VectorSubcoreMesh(core_axis_name="c", subcore_axis_name="s"),
        compiler_params=pltpu.CompilerParams(needs_layout_passes=False),
    )
    hist96 = sc_scatter(idx96, w96)

    hist32 = hist96.reshape(B, C * P2)

    out = pl.pallas_call(
        _norm_mlp_kernel,
        out_shape=jax.ShapeDtypeStruct((B, W3.shape[0]), jnp.float32),
        compiler_params=pltpu.CompilerParams(
            vmem_limit_bytes=100 * 1024 * 1024,
        ),
    )(hist32, W1, b1, W2, b2, W3, b3)
    return out[:, :, None, None]


# speedup vs baseline: 7.3753x; 1.3944x over previous
import jax
import jax.numpy as jnp
from jax import lax
from jax.experimental import pallas as pl
from jax.experimental.pallas import tpu as pltpu
from jax.experimental.pallas import tpu_sc as plsc
from functools import partial

N_BINS = 32
EPS = 6.4 / 256
LOW = -3.2 - EPS / 2
HIGH = 3.2 - EPS / 2
WIDTH = HIGH - LOW

NUM_WORKERS = 32
ROWS_PER_WORKER = 3


def _bin_one(si, su, sv, base_w):
    iu = jnp.log(si / su)
    iv = jnp.log(si / sv)
    bu = jnp.floor((iu - LOW) / WIDTH * N_BINS).astype(jnp.int32)
    bv = jnp.floor((iv - LOW) / WIDTH * N_BINS).astype(jnp.int32)
    bu = jnp.where(iu == HIGH, N_BINS - 1, bu)
    bv = jnp.where(iv == HIGH, N_BINS - 1, bv)
    in_u = ((iu >= LOW) & (iu <= HIGH) & (bu >= 0) & (bu < N_BINS))
    in_v = ((iv >= LOW) & (iv <= HIGH) & (bv >= 0) & (bv < N_BINS))
    w = base_w * in_u.astype(jnp.float32) * in_v.astype(jnp.float32)
    idx = jnp.where(w > 0, bu * N_BINS + bv, 0)
    return idx, w


def _gather_bin_kernel(*refs):
    idx_ref, w_ref = refs[12], refs[13]
    rows = lax.broadcasted_iota(jnp.int32, (4 * 512, 4 * N_BINS), 0)
    cols = lax.broadcasted_iota(jnp.int32, (4 * 512, 4 * N_BINS), 1)
    sel = (rows == (cols // N_BINS) * 512 +
           (cols % N_BINS) * 16).astype(jnp.float32)

    p = []
    for c in range(3):
        xc = jnp.concatenate(
            [refs[4 * c + k][:, 0, 0, :] for k in range(4)], axis=1)
        p.append(lax.dot_general(xc, sel, (((1,), (0,)), ((), ())),
                                 preferred_element_type=jnp.float32,
                                 precision=lax.Precision.HIGHEST))
    p0, p1, p2 = p

    valid = ((p0 > 0) & (p1 > 0) & (p2 > 0)).astype(jnp.float32)
    iy = jnp.sqrt(p0 * p0 + p1 * p1 + p2 * p2)
    s0 = jnp.where(p0 > 0, p0, 1.0)
    s1 = jnp.where(p1 > 0, p1, 1.0)
    s2 = jnp.where(p2 > 0, p2, 1.0)
    base_w = iy * valid

    i0, w0 = _bin_one(s0, s2, s1, base_w)
    i1, w1 = _bin_one(s1, s2, s0, base_w)
    i2, w2 = _bin_one(s2, s1, s0, base_w)

    idx_ref[...] = jnp.stack([i0, i1, i2], axis=1).reshape(96, 4 * N_BINS)
    w_ref[...] = jnp.stack([w0, w1, w2], axis=1).reshape(96, 4 * N_BINS)


def _sc_scatter_body(idx_hbm, w_hbm, out_hbm, idx_v, w_v, hist_v):
    c = lax.axis_index("c")
    s = lax.axis_index("s")
    P2 = N_BINS * N_BINS

    @pl.when(s < 6)
    def _():
        base = (c * 6 + s) * 8
        pltpu.sync_copy(idx_hbm.at[pl.ds(base, 8), :], idx_v)
        pltpu.sync_copy(w_hbm.at[pl.ds(base, 8), :], w_v)

        zeros = jnp.zeros((16,), jnp.float32)

        def zero_body(k, carry):
            for j in range(8):
                hist_v[j, pl.ds(k * 16, 16)] = zeros
            return carry

        lax.fori_loop(0, P2 // 16, zero_body, 0)

        def scatter_body(k, carry):
            j = k // (P2 // 16)
            rowv = jnp.full((16,), j, jnp.int32)
            iv = idx_v[j, pl.ds((k % 64) * 16, 16)]
            wv = w_v[j, pl.ds((k % 64) * 16, 16)]
            plsc.addupdate_scatter(hist_v, [rowv, iv], wv)
            return carry

        lax.fori_loop(0, 8 * P2 // 16, scatter_body, 0)

        pltpu.sync_copy(hist_v, out_hbm.at[pl.ds(base, 8), :])


def _norm_mlp_kernel(h_ref, w1_ref, b1_ref, w2_ref, b2_ref, w3_ref, b3_ref,
                     out_ref):
    h = h_ref[...]
    P2 = N_BINS * N_BINS
    parts = []
    for c in range(3):
        hc = h[:, c * P2:(c + 1) * P2]
        norm = jnp.sum(hc, axis=1, keepdims=True)
        parts.append(jnp.sqrt(hc / norm))
    hist = jnp.concatenate(parts, axis=1)

    h1 = lax.dot_general(hist, w1_ref[...], (((1,), (1,)), ((), ())),
                         preferred_element_type=jnp.float32)
    h1 = jnp.maximum(h1 + b1_ref[...][None, :], 0.0)
    h2 = lax.dot_general(h1, w2_ref[...], (((1,), (1,)), ((), ())),
                         preferred_element_type=jnp.float32)
    h2 = jnp.maximum(h2 + b2_ref[...][None, :], 0.0)
    h3 = lax.dot_general(h2, w3_ref[...], (((1,), (1,)), ((), ())),
                         preferred_element_type=jnp.float32)
    out_ref[...] = jnp.maximum(h3 + b3_ref[...][None, :], 0.0)


@jax.jit
def kernel(inp_img, W1, b1, W2, b2, W3, b3):
    B, C, H, W = inp_img.shape
    BC = B * C
    x3 = inp_img.reshape(BC, H, W)

    P2 = N_BINS * N_BINS
    in_specs = [
        pl.BlockSpec((B, 1, 8, W),
                     lambda i, c=c, k=k: (0, c, 2 * (4 * i + k), 0))
        for c in range(3) for k in range(4)
    ]
    idx96, w96 = pl.pallas_call(
        _gather_bin_kernel,
        grid=(8,),
        in_specs=in_specs,
        out_specs=[
            pl.BlockSpec((BC, 4 * N_BINS), lambda i: (0, i)),
            pl.BlockSpec((BC, 4 * N_BINS), lambda i: (0, i)),
        ],
        out_shape=[
            jax.ShapeDtypeStruct((BC, P2), jnp.int32),
            jax.ShapeDtypeStruct((BC, P2), jnp.float32),
        ],
    )(*([inp_img] * 12))

    sc_scatter = pl.kernel(
        _sc_scatter_body,
        out_type=jax.ShapeDtypeStruct((BC, P2), jnp.float32),
        scratch_types=[
            pltpu.VMEM((8, P2), jnp.int32),
            pltpu.VMEM((8, P2), jnp.float32),
            pltpu.VMEM((8, P2), jnp.float32),
        ],
        mesh=plsc.VectorSubcoreMesh(core_axis_name="c", subcore_axis_name="s"),
        compiler_params=pltpu.CompilerParams(needs_layout_passes=False),
    )
    hist96 = sc_scatter(idx96, w96)

    hist32 = hist96.reshape(B, C * P2)

    out = pl.pallas_call(
        _norm_mlp_kernel,
        out_shape=jax.ShapeDtypeStruct((B, W3.shape[0]), jnp.float32),
        compiler_params=pltpu.CompilerParams(
            vmem_limit_bytes=100 * 1024 * 1024,
        ),
    )(hist32, W1, b1, W2, b2, W3, b3)
    return out[:, :, None, None]
